# baseline (speedup 1.0000x reference)
import jax
import jax.numpy as jnp
from jax import lax
from jax.experimental import pallas as pl
from jax.experimental.pallas import tpu as pltpu
from jax.experimental.pallas import tpu_sc as plsc

D_SINGLE = 384
D_PAIR = 128
NUM_AA = 21
MAX_REL = 32
N_RES = 1024

BI = 8
PAD_ROWS = 2 * N_RES + 2 * MAX_REL


def _pair_kernel(table_ref, out_ref, padded_ref):
    blk = pl.program_id(0)

    @pl.when(blk == 0)
    def _build_padded():
        hi = table_ref[2 * MAX_REL, :]
        lo = table_ref[0, :]
        padded_ref[pl.ds(0, N_RES), :] = jnp.broadcast_to(hi, (N_RES, D_PAIR))
        for r in range(2 * MAX_REL + 1):
            padded_ref[N_RES + r, :] = table_ref[2 * MAX_REL - r, :]
        tail = PAD_ROWS - N_RES - 2 * MAX_REL - 1
        padded_ref[pl.ds(N_RES + 2 * MAX_REL + 1, tail), :] = jnp.broadcast_to(
            lo, (tail, D_PAIR)
        )

    for k in range(BI):
        i = blk * BI + k
        start = (N_RES + MAX_REL) - i
        out_ref[k] = padded_ref[pl.ds(start, N_RES), :]


_N_SC_WORKERS = 32
_ROWS_PER_W = N_RES // _N_SC_WORKERS


def _single_sc_kernel(table_hbm, idx_hbm, out_hbm, idx_v, rows_v, sem):
    wid = lax.axis_index("s") * 2 + lax.axis_index("c")
    base = wid * _ROWS_PER_W
    pltpu.sync_copy(idx_hbm.at[pl.ds(base, _ROWS_PER_W)], idx_v)
    pltpu.async_copy(table_hbm.at[idx_v], rows_v, sem).wait()
    pltpu.sync_copy(rows_v, out_hbm.at[pl.ds(base, _ROWS_PER_W)])


def kernel(target_seq, W_dense, b_dense, relpos_table):
    pair = pl.pallas_call(
        _pair_kernel,
        grid=(N_RES // BI,),
        in_specs=[pl.BlockSpec((2 * MAX_REL + 1, D_PAIR), lambda b: (0, 0))],
        out_specs=pl.BlockSpec((BI, N_RES, D_PAIR), lambda b: (b, 0, 0)),
        out_shape=jax.ShapeDtypeStruct((N_RES, N_RES, D_PAIR), jnp.float32),
        scratch_shapes=[pltpu.VMEM((PAD_ROWS, D_PAIR), jnp.float32)],
    )(relpos_table)

    table_wb = W_dense + b_dense[None, :]
    single = pl.kernel(
        _single_sc_kernel,
        mesh=plsc.VectorSubcoreMesh(core_axis_name="c", subcore_axis_name="s"),
        out_type=jax.ShapeDtypeStruct((N_RES, D_SINGLE), jnp.float32),
        scratch_types=[
            pltpu.VMEM((_ROWS_PER_W,), jnp.int32),
            pltpu.VMEM((_ROWS_PER_W, D_SINGLE), jnp.float32),
            pltpu.SemaphoreType.DMA,
        ],
    )(table_wb, target_seq.astype(jnp.int32))

    return (single, pair)
